# baseline (speedup 1.0000x reference)
import functools

import jax
import jax.numpy as jnp
from jax import lax
from jax.experimental import pallas as pl
from jax.experimental.pallas import tpu as pltpu
from jax.experimental.pallas import tpu_sc as plsc

N = 320000
D = 128
NSEG = 10000
EPS = 1e-5
NEG = float("-inf")

NWORK = 32
SEGW = 320
OFF_PAD = 10256
ROWS_PER_WORKER = N // NWORK
CHUNKS_PER_WORKER = ROWS_PER_WORKER // 16
GCH = 80
LCAP = 10016
UPAD = 10240

_SC_MESH = dict(core_axis_name="c", subcore_axis_name="s")


def _wid():
    return lax.axis_index("s") * 2 + lax.axis_index("c")


def _ln_relu(y, g, be):
    mu = jnp.mean(y, axis=-1, keepdims=True)
    d = y - mu
    var = jnp.mean(d * d, axis=-1, keepdims=True)
    return jnp.maximum(d * lax.rsqrt(var + EPS) * g + be, 0.0)


def _enc0_body(x_ref, w_ref, b_ref, g_ref, be_ref, o_ref):
    y = jnp.dot(x_ref[...], w_ref[...], preferred_element_type=jnp.float32)
    o_ref[...] = _ln_relu(y + b_ref[...], g_ref[...], be_ref[...])


def _enc2_body(h_ref, a_ref, w_ref, g_ref, be_ref, o_ref):
    y = jnp.dot(h_ref[...], w_ref[...], preferred_element_type=jnp.float32)
    o_ref[...] = _ln_relu(y + a_ref[...], g_ref[...], be_ref[...])


def _tiny_body(s_ref, w_ref, b_ref, o_ref):
    o_ref[...] = (
        jnp.dot(s_ref[...], w_ref[...], preferred_element_type=jnp.float32)
        + b_ref[...]
    )


_ROWBLK = 1280


def _enc0(x, w, b, g, be):
    return pl.pallas_call(
        _enc0_body,
        grid=(N // _ROWBLK,),
        in_specs=[
            pl.BlockSpec((_ROWBLK, D), lambda i: (i, 0)),
            pl.BlockSpec((D, D), lambda i: (0, 0)),
            pl.BlockSpec((1, D), lambda i: (0, 0)),
            pl.BlockSpec((1, D), lambda i: (0, 0)),
            pl.BlockSpec((1, D), lambda i: (0, 0)),
        ],
        out_specs=pl.BlockSpec((_ROWBLK, D), lambda i: (i, 0)),
        out_shape=jax.ShapeDtypeStruct((N, D), jnp.float32),
    )(x, w, b.reshape(1, D), g.reshape(1, D), be.reshape(1, D))


def _enc2(h, a, w, g, be):
    return pl.pallas_call(
        _enc2_body,
        grid=(N // _ROWBLK,),
        in_specs=[
            pl.BlockSpec((_ROWBLK, D), lambda i: (i, 0)),
            pl.BlockSpec((_ROWBLK, D), lambda i: (i, 0)),
            pl.BlockSpec((D, D), lambda i: (0, 0)),
            pl.BlockSpec((1, D), lambda i: (0, 0)),
            pl.BlockSpec((1, D), lambda i: (0, 0)),
        ],
        out_specs=pl.BlockSpec((_ROWBLK, D), lambda i: (i, 0)),
        out_shape=jax.ShapeDtypeStruct((N, D), jnp.float32),
    )(h, a, w, g.reshape(1, D), be.reshape(1, D))


def _tiny(sm, w, b):
    return pl.pallas_call(
        _tiny_body,
        grid=(5,),
        in_specs=[
            pl.BlockSpec((NSEG // 5, D), lambda i: (i, 0)),
            pl.BlockSpec((D, D), lambda i: (0, 0)),
            pl.BlockSpec((1, D), lambda i: (0, 0)),
        ],
        out_specs=pl.BlockSpec((NSEG // 5, D), lambda i: (i, 0)),
        out_shape=jax.ShapeDtypeStruct((NSEG, D), jnp.float32),
    )(sm, w, b.reshape(1, D))


def _scan_kernel(ids_hbm, sidx_hbm, spos_hbm, cnt_hbm,
                 buf24, li_v, lp_v, cnt_v):
    wid = _wid()
    base = wid * ROWS_PER_WORKER
    iota = lax.iota(jnp.int32, 16)

    def chunk(t, pos):
        src = base + 16 * t
        pltpu.sync_copy(ids_hbm.at[pl.ds(src, 24)], buf24)
        cur = buf24[pl.ds(8, 16)]
        prev = buf24[pl.ds(7, 16)]
        mi = jnp.minimum(jnp.abs(cur - prev), 1)
        pre = jnp.cumsum(mi)
        gpos = jnp.full((16,), src, jnp.int32) + iota
        tr = LCAP + iota
        dest = tr + mi * (pos + pre - 1 - tr)
        plsc.store_scatter(li_v, [dest], cur)
        plsc.store_scatter(lp_v, [dest], gpos)
        return pos + pre[15]

    pos = lax.fori_loop(0, CHUNKS_PER_WORKER, chunk, jnp.int32(0))
    pltpu.sync_copy(li_v.at[pl.ds(0, LCAP)], sidx_hbm.at[pl.ds(wid * LCAP, LCAP)])
    pltpu.sync_copy(lp_v.at[pl.ds(0, LCAP)], spos_hbm.at[pl.ds(wid * LCAP, LCAP)])
    cnt_v[...] = jnp.full((16,), pos, jnp.int32)
    pltpu.sync_copy(cnt_v, cnt_hbm.at[pl.ds(wid * 16, 16)])


def _sc_scan(ids_pad):
    fn = functools.partial(
        pl.kernel,
        out_type=(
            jax.ShapeDtypeStruct((NWORK * LCAP,), jnp.int32),
            jax.ShapeDtypeStruct((NWORK * LCAP,), jnp.int32),
            jax.ShapeDtypeStruct((NWORK * 16,), jnp.int32),
        ),
        mesh=plsc.VectorSubcoreMesh(**_SC_MESH),
        scratch_types=[
            pltpu.VMEM((24,), jnp.int32),
            pltpu.VMEM((LCAP + 16,), jnp.int32),
            pltpu.VMEM((LCAP + 16,), jnp.int32),
            pltpu.VMEM((16,), jnp.int32),
        ],
        compiler_params=pltpu.CompilerParams(needs_layout_passes=False),
    )(_scan_kernel)
    return fn(ids_pad)


def _merge_kernel(sidx_hbm, spos_hbm, cnt_hbm, offp_hbm, uniq_hbm,
                  off_v, uniq_v, buf_i, buf_p, cnt_v):
    wid = _wid()
    iota = lax.iota(jnp.int32, 16)

    @pl.when(wid == 0)
    def _():
        sentinel = jnp.full((16,), N, jnp.int32)
        for t in range(OFF_PAD // 16):
            off_v[pl.ds(16 * t, 16)] = sentinel
        pltpu.sync_copy(cnt_hbm, cnt_v)

        for w in range(NWORK):
            cw = cnt_v[pl.ds(w * 16, 16)][0]
            pltpu.sync_copy(sidx_hbm.at[pl.ds(w * LCAP, LCAP)], buf_i)
            pltpu.sync_copy(spos_hbm.at[pl.ds(w * LCAP, LCAP)], buf_p)

            def scat(t, _, cw=cw):
                m = (16 * t + iota) < cw
                idxv = buf_i[pl.ds(16 * t, 16)]
                posv = buf_p[pl.ds(16 * t, 16)]
                dest = jnp.where(m, idxv, OFF_PAD - 16 + iota)
                plsc.store_scatter(off_v, [dest], posv)
                return 0

            lax.fori_loop(0, (cw + 15) // 16, scat, 0)

        off_v[pl.ds(OFF_PAD - 16, 16)] = sentinel

        def back(j, carry):
            t = OFF_PAD // 16 - 1 - j
            v = off_v[pl.ds(16 * t, 16)]
            rv = lax.rev(v, (0,))
            runmin = -plsc.cummax(-rv)
            suff = jnp.minimum(lax.rev(runmin, (0,)),
                               jnp.full((16,), carry, jnp.int32))
            off_v[pl.ds(16 * t, 16)] = suff
            return suff[0]

        lax.fori_loop(0, OFF_PAD // 16, back, jnp.int32(N))

        def uniq_step(t, pos):
            a = off_v[pl.ds(16 * t, 16)]
            b = off_v[pl.ds(16 * t + 1, 16)]
            mi = (b > a).astype(jnp.int32)
            pre = jnp.cumsum(mi)
            svec = jnp.full((16,), 16 * t, jnp.int32) + iota
            dest = jnp.where(mi == 1, pos + pre - 1, UPAD + iota)
            plsc.store_scatter(uniq_v, [dest], svec)
            return pos + pre[15]

        pos = lax.fori_loop(0, NSEG // 16, uniq_step, jnp.int32(0))
        u0 = jnp.full((16,), uniq_v[pl.ds(0, 16)][0], jnp.int32)

        def fill(t, _):
            ex = uniq_v[pl.ds(16 * t, 16)]
            lm = (16 * t + iota) >= pos
            uniq_v[pl.ds(16 * t, 16)] = jnp.where(lm, u0, ex)
            return 0

        lax.fori_loop(0, UPAD // 16, fill, 0)

        pltpu.sync_copy(off_v, offp_hbm)
        pltpu.sync_copy(uniq_v.at[pl.ds(0, UPAD)], uniq_hbm)


def _sc_merge(sidx, spos, cnt):
    fn = functools.partial(
        pl.kernel,
        out_type=(
            jax.ShapeDtypeStruct((OFF_PAD,), jnp.int32),
            jax.ShapeDtypeStruct((UPAD,), jnp.int32),
        ),
        mesh=plsc.VectorSubcoreMesh(**_SC_MESH),
        scratch_types=[
            pltpu.VMEM((OFF_PAD,), jnp.int32),
            pltpu.VMEM((UPAD + 16,), jnp.int32),
            pltpu.VMEM((LCAP,), jnp.int32),
            pltpu.VMEM((LCAP,), jnp.int32),
            pltpu.VMEM((NWORK * 16,), jnp.int32),
        ],
        compiler_params=pltpu.CompilerParams(needs_layout_passes=False),
    )(_merge_kernel)
    return fn(sidx, spos, cnt)


CROWS = 40


def _segmax_kernel(h_hbm, off_hbm, sm_hbm, off_v, res_v,
                   buf_a, buf_b, buf_t, sem_a, sem_b):
    wid = _wid()
    lo = wid * SEGW
    pltpu.sync_copy(off_hbm.at[pl.ds(lo, 328)], off_v.at[pl.ds(0, 328)])
    neg = jnp.full((16,), NEG, jnp.float32)

    def chunk0_addr(start):
        a0 = (start // 8) * 8
        return pl.multiple_of(jnp.minimum(a0, N - CROWS), 8)

    def process(i, buf, cb, rlow, end):
        valids = []
        for j in range(CROWS):
            gr = cb + j
            valids.append(jnp.logical_and(gr >= rlow, gr < end))
        for kk in range(8):
            cur = res_v[i, pl.ds(kk * 16, 16)]
            for j in range(CROWS):
                v = buf[j, pl.ds(kk * 16, 16)]
                cur = jnp.maximum(cur, jnp.where(valids[j], v, NEG))
            res_v[i, pl.ds(kk * 16, 16)] = cur

    def seg_body(i, mybuf, mysem, nxtbuf, nxtsem):
        ovec = off_v[pl.ds(i, 16)]
        start = ovec[0]
        end = ovec[1]
        nstart = off_v[pl.ds(i + 1, 16)][0]

        @pl.when(i + 1 < SEGW)
        def _():
            pltpu.async_copy(
                h_hbm.at[pl.ds(chunk0_addr(nstart), CROWS)], nxtbuf, nxtsem
            )

        for kk in range(8):
            res_v[i, pl.ds(kk * 16, 16)] = neg
        a0 = (start // 8) * 8
        cb0 = chunk0_addr(start)
        nchunk = (end - a0 + CROWS - 1) // CROWS
        pltpu.make_async_copy(
            h_hbm.at[pl.ds(0, CROWS)], mybuf, mysem
        ).wait()
        process(i, mybuf, cb0, start, end)

        def tail(t, _):
            pos = a0 + CROWS * t
            cb = pl.multiple_of(jnp.minimum(pos, N - CROWS), 8)
            pltpu.sync_copy(h_hbm.at[pl.ds(cb, CROWS)], buf_t)
            process(i, buf_t, cb, jnp.maximum(start, pos), end)
            return 0

        lax.fori_loop(1, nchunk, tail, 0)

    s0 = off_v[pl.ds(0, 16)][0]
    pltpu.async_copy(h_hbm.at[pl.ds(chunk0_addr(s0), CROWS)], buf_a, sem_a)

    def pair(p, _):
        seg_body(2 * p, buf_a, sem_a, buf_b, sem_b)
        seg_body(2 * p + 1, buf_b, sem_b, buf_a, sem_a)
        return 0

    lax.fori_loop(0, SEGW // 2, pair, 0)

    @pl.when(wid < NWORK - 1)
    def _():
        pltpu.sync_copy(res_v, sm_hbm.at[pl.ds(lo, SEGW)])

    @pl.when(wid == NWORK - 1)
    def _():
        pltpu.sync_copy(res_v.at[pl.ds(0, 80)], sm_hbm.at[pl.ds(lo, 80)])


def _sc_segmax(h, off_padded):
    fn = functools.partial(
        pl.kernel,
        out_type=jax.ShapeDtypeStruct((NSEG, D), jnp.float32),
        mesh=plsc.VectorSubcoreMesh(**_SC_MESH),
        scratch_types=[
            pltpu.VMEM((344,), jnp.int32),
            pltpu.VMEM((SEGW, D), jnp.float32),
            pltpu.VMEM((CROWS, D), jnp.float32),
            pltpu.VMEM((CROWS, D), jnp.float32),
            pltpu.VMEM((CROWS, D), jnp.float32),
            pltpu.SemaphoreType.DMA,
            pltpu.SemaphoreType.DMA,
        ],
    )(_segmax_kernel)
    return fn(h, off_padded)


def _make_gather(nrows, idx_off):
    rpw = nrows // NWORK
    n = rpw // GCH

    def gather_kernel(tab_hbm, ids_hbm, out_hbm, ids_v,
                      rows_a, rows_b, sem_a, sem_b):
        wid = _wid()
        base = wid * rpw
        pltpu.sync_copy(ids_hbm.at[pl.ds(base + idx_off, rpw)], ids_v)

        def issue(t, buf, sem):
            pltpu.async_copy(tab_hbm.at[ids_v.at[pl.ds(t * GCH, GCH)]],
                             buf, sem)

        def wait(buf, sem):
            pltpu.make_async_copy(
                tab_hbm.at[ids_v.at[pl.ds(0, GCH)]], buf, sem
            ).wait()

        issue(0, rows_a, sem_a)

        def pair(p, _):
            t0 = 2 * p
            t1 = t0 + 1
            wait(rows_a, sem_a)

            @pl.when(t1 < n)
            def _():
                issue(t1, rows_b, sem_b)

            pltpu.sync_copy(rows_a, out_hbm.at[pl.ds(base + t0 * GCH, GCH)])

            @pl.when(t1 < n)
            def _():
                @pl.when(t1 + 1 < n)
                def _():
                    issue(t1 + 1, rows_a, sem_a)

                wait(rows_b, sem_b)
                pltpu.sync_copy(rows_b,
                                out_hbm.at[pl.ds(base + t1 * GCH, GCH)])

            return 0

        lax.fori_loop(0, (n + 1) // 2, pair, 0)

    def run(tab, ids):
        fn = functools.partial(
            pl.kernel,
            out_type=jax.ShapeDtypeStruct((nrows, D), jnp.float32),
            mesh=plsc.VectorSubcoreMesh(**_SC_MESH),
            scratch_types=[
                pltpu.VMEM((rpw,), jnp.int32),
                pltpu.VMEM((GCH, D), jnp.float32),
                pltpu.VMEM((GCH, D), jnp.float32),
                pltpu.SemaphoreType.DMA,
                pltpu.SemaphoreType.DMA,
            ],
        )(gather_kernel)
        return fn(tab, ids)

    return run


_gather_rows = _make_gather(N, 8)
_gather_uniq = _make_gather(UPAD, 0)


def kernel(x, polyline_ids, W0, b0, g0, be0, W1, b1, g1, be1, W2, b2, g2, be2):
    ids = polyline_ids.astype(jnp.int32)
    ids_pad = jnp.concatenate([jnp.full((8,), -1, jnp.int32), ids])

    sidx, spos, cnt = _sc_scan(ids_pad)
    off_padded, uniq_pad = _sc_merge(sidx, spos, cnt)

    h0 = _enc0(x, W0, b0, g0, be0)
    sm0 = _sc_segmax(h0, off_padded)
    c1 = _tiny(sm0, W1[D:], b1)
    a1 = _gather_rows(c1, ids_pad)
    h1 = _enc2(h0, a1, W1[:D], g1, be1)
    sm1 = _sc_segmax(h1, off_padded)
    c2 = _tiny(sm1, W2[D:], b2)
    a2 = _gather_rows(c2, ids_pad)
    h2 = _enc2(h1, a2, W2[:D], g2, be2)
    smf = _sc_segmax(h2, off_padded)

    feats_pad = _gather_uniq(smf, uniq_pad)
    return (feats_pad[:NSEG], uniq_pad[:NSEG])


# speedup vs baseline: 1.1802x; 1.1802x over previous
import functools

import jax
import jax.numpy as jnp
from jax import lax
from jax.experimental import pallas as pl
from jax.experimental.pallas import tpu as pltpu
from jax.experimental.pallas import tpu_sc as plsc

N = 320000
D = 128
NSEG = 10000
EPS = 1e-5
NEG = float("-inf")

NWORK = 32
SEGW = 320
OFF_PAD = 10256
ROWS_PER_WORKER = N // NWORK
CHUNKS_PER_WORKER = ROWS_PER_WORKER // 16
GCH = 80
LCAP = 10016
UPAD = 10240

_SC_MESH = dict(core_axis_name="c", subcore_axis_name="s")


def _wid():
    return lax.axis_index("s") * 2 + lax.axis_index("c")


def _ln_relu(y, g, be):
    mu = jnp.mean(y, axis=-1, keepdims=True)
    d = y - mu
    var = jnp.mean(d * d, axis=-1, keepdims=True)
    return jnp.maximum(d * lax.rsqrt(var + EPS) * g + be, 0.0)


def _enc0_body(x_ref, w_ref, b_ref, g_ref, be_ref, o_ref):
    y = jnp.dot(x_ref[...], w_ref[...], preferred_element_type=jnp.float32)
    o_ref[...] = _ln_relu(y + b_ref[...], g_ref[...], be_ref[...])


def _enc2_body(h_ref, a_ref, w_ref, g_ref, be_ref, o_ref):
    y = jnp.dot(h_ref[...], w_ref[...], preferred_element_type=jnp.float32)
    o_ref[...] = _ln_relu(y + a_ref[...], g_ref[...], be_ref[...])


def _tiny_body(s_ref, w_ref, b_ref, o_ref):
    o_ref[...] = (
        jnp.dot(s_ref[...], w_ref[...], preferred_element_type=jnp.float32)
        + b_ref[...]
    )


_ROWBLK = 1280


def _enc0(x, w, b, g, be):
    return pl.pallas_call(
        _enc0_body,
        grid=(N // _ROWBLK,),
        in_specs=[
            pl.BlockSpec((_ROWBLK, D), lambda i: (i, 0)),
            pl.BlockSpec((D, D), lambda i: (0, 0)),
            pl.BlockSpec((1, D), lambda i: (0, 0)),
            pl.BlockSpec((1, D), lambda i: (0, 0)),
            pl.BlockSpec((1, D), lambda i: (0, 0)),
        ],
        out_specs=pl.BlockSpec((_ROWBLK, D), lambda i: (i, 0)),
        out_shape=jax.ShapeDtypeStruct((N, D), jnp.float32),
    )(x, w, b.reshape(1, D), g.reshape(1, D), be.reshape(1, D))


def _enc2(h, a, w, g, be):
    return pl.pallas_call(
        _enc2_body,
        grid=(N // _ROWBLK,),
        in_specs=[
            pl.BlockSpec((_ROWBLK, D), lambda i: (i, 0)),
            pl.BlockSpec((_ROWBLK, D), lambda i: (i, 0)),
            pl.BlockSpec((D, D), lambda i: (0, 0)),
            pl.BlockSpec((1, D), lambda i: (0, 0)),
            pl.BlockSpec((1, D), lambda i: (0, 0)),
        ],
        out_specs=pl.BlockSpec((_ROWBLK, D), lambda i: (i, 0)),
        out_shape=jax.ShapeDtypeStruct((N, D), jnp.float32),
    )(h, a, w, g.reshape(1, D), be.reshape(1, D))


def _tiny(sm, w, b):
    return pl.pallas_call(
        _tiny_body,
        grid=(5,),
        in_specs=[
            pl.BlockSpec((NSEG // 5, D), lambda i: (i, 0)),
            pl.BlockSpec((D, D), lambda i: (0, 0)),
            pl.BlockSpec((1, D), lambda i: (0, 0)),
        ],
        out_specs=pl.BlockSpec((NSEG // 5, D), lambda i: (i, 0)),
        out_shape=jax.ShapeDtypeStruct((NSEG, D), jnp.float32),
    )(sm, w, b.reshape(1, D))


def _scan_kernel(ids_hbm, sidx_hbm, spos_hbm, cnt_hbm,
                 buf24, li_v, lp_v, cnt_v):
    wid = _wid()
    base = wid * ROWS_PER_WORKER
    iota = lax.iota(jnp.int32, 16)

    def chunk(t, pos):
        src = base + 16 * t
        pltpu.sync_copy(ids_hbm.at[pl.ds(src, 24)], buf24)
        cur = buf24[pl.ds(8, 16)]
        prev = buf24[pl.ds(7, 16)]
        mi = jnp.minimum(jnp.abs(cur - prev), 1)
        pre = jnp.cumsum(mi)
        gpos = jnp.full((16,), src, jnp.int32) + iota
        tr = LCAP + iota
        dest = tr + mi * (pos + pre - 1 - tr)
        plsc.store_scatter(li_v, [dest], cur)
        plsc.store_scatter(lp_v, [dest], gpos)
        return pos + pre[15]

    pos = lax.fori_loop(0, CHUNKS_PER_WORKER, chunk, jnp.int32(0))
    pltpu.sync_copy(li_v.at[pl.ds(0, LCAP)], sidx_hbm.at[pl.ds(wid * LCAP, LCAP)])
    pltpu.sync_copy(lp_v.at[pl.ds(0, LCAP)], spos_hbm.at[pl.ds(wid * LCAP, LCAP)])
    cnt_v[...] = jnp.full((16,), pos, jnp.int32)
    pltpu.sync_copy(cnt_v, cnt_hbm.at[pl.ds(wid * 16, 16)])


def _sc_scan(ids_pad):
    fn = functools.partial(
        pl.kernel,
        out_type=(
            jax.ShapeDtypeStruct((NWORK * LCAP,), jnp.int32),
            jax.ShapeDtypeStruct((NWORK * LCAP,), jnp.int32),
            jax.ShapeDtypeStruct((NWORK * 16,), jnp.int32),
        ),
        mesh=plsc.VectorSubcoreMesh(**_SC_MESH),
        scratch_types=[
            pltpu.VMEM((24,), jnp.int32),
            pltpu.VMEM((LCAP + 16,), jnp.int32),
            pltpu.VMEM((LCAP + 16,), jnp.int32),
            pltpu.VMEM((16,), jnp.int32),
        ],
        compiler_params=pltpu.CompilerParams(needs_layout_passes=False),
    )(_scan_kernel)
    return fn(ids_pad)


def _merge_kernel(sidx_hbm, spos_hbm, cnt_hbm, offp_hbm, uniq_hbm,
                  off_v, uniq_v, buf_i, buf_p, cnt_v):
    wid = _wid()
    iota = lax.iota(jnp.int32, 16)

    @pl.when(wid == 0)
    def _():
        sentinel = jnp.full((16,), N, jnp.int32)
        for t in range(OFF_PAD // 16):
            off_v[pl.ds(16 * t, 16)] = sentinel
        pltpu.sync_copy(cnt_hbm, cnt_v)

        for w in range(NWORK):
            cw = cnt_v[pl.ds(w * 16, 16)][0]
            pltpu.sync_copy(sidx_hbm.at[pl.ds(w * LCAP, LCAP)], buf_i)
            pltpu.sync_copy(spos_hbm.at[pl.ds(w * LCAP, LCAP)], buf_p)

            def scat(t, _, cw=cw):
                m = (16 * t + iota) < cw
                idxv = buf_i[pl.ds(16 * t, 16)]
                posv = buf_p[pl.ds(16 * t, 16)]
                dest = jnp.where(m, idxv, OFF_PAD - 16 + iota)
                plsc.store_scatter(off_v, [dest], posv)
                return 0

            lax.fori_loop(0, (cw + 15) // 16, scat, 0)

        off_v[pl.ds(OFF_PAD - 16, 16)] = sentinel

        def back(j, carry):
            t = OFF_PAD // 16 - 1 - j
            v = off_v[pl.ds(16 * t, 16)]
            rv = lax.rev(v, (0,))
            runmin = -plsc.cummax(-rv)
            suff = jnp.minimum(lax.rev(runmin, (0,)),
                               jnp.full((16,), carry, jnp.int32))
            off_v[pl.ds(16 * t, 16)] = suff
            return suff[0]

        lax.fori_loop(0, OFF_PAD // 16, back, jnp.int32(N))

        def uniq_step(t, pos):
            a = off_v[pl.ds(16 * t, 16)]
            b = off_v[pl.ds(16 * t + 1, 16)]
            mi = (b > a).astype(jnp.int32)
            pre = jnp.cumsum(mi)
            svec = jnp.full((16,), 16 * t, jnp.int32) + iota
            dest = jnp.where(mi == 1, pos + pre - 1, UPAD + iota)
            plsc.store_scatter(uniq_v, [dest], svec)
            return pos + pre[15]

        pos = lax.fori_loop(0, NSEG // 16, uniq_step, jnp.int32(0))
        u0 = jnp.full((16,), uniq_v[pl.ds(0, 16)][0], jnp.int32)

        def fill(t, _):
            ex = uniq_v[pl.ds(16 * t, 16)]
            lm = (16 * t + iota) >= pos
            uniq_v[pl.ds(16 * t, 16)] = jnp.where(lm, u0, ex)
            return 0

        lax.fori_loop(0, UPAD // 16, fill, 0)

        pltpu.sync_copy(off_v, offp_hbm)
        pltpu.sync_copy(uniq_v.at[pl.ds(0, UPAD)], uniq_hbm)


def _sc_merge(sidx, spos, cnt):
    fn = functools.partial(
        pl.kernel,
        out_type=(
            jax.ShapeDtypeStruct((OFF_PAD,), jnp.int32),
            jax.ShapeDtypeStruct((UPAD,), jnp.int32),
        ),
        mesh=plsc.VectorSubcoreMesh(**_SC_MESH),
        scratch_types=[
            pltpu.VMEM((OFF_PAD,), jnp.int32),
            pltpu.VMEM((UPAD + 16,), jnp.int32),
            pltpu.VMEM((LCAP,), jnp.int32),
            pltpu.VMEM((LCAP,), jnp.int32),
            pltpu.VMEM((NWORK * 16,), jnp.int32),
        ],
        compiler_params=pltpu.CompilerParams(needs_layout_passes=False),
    )(_merge_kernel)
    return fn(sidx, spos, cnt)


CROWS = 48


def _segmax_kernel(h_hbm, off_hbm, sm_hbm, off_v, res_v,
                   buf_a, buf_b, buf_t, sem_a, sem_b):
    wid = _wid()
    lo = wid * SEGW
    pltpu.sync_copy(off_hbm.at[pl.ds(lo, 328)], off_v.at[pl.ds(0, 328)])
    neg = jnp.full((16,), NEG, jnp.float32)

    def chunk0_addr(start):
        a0 = (start // 8) * 8
        return pl.multiple_of(jnp.minimum(a0, N - CROWS), 8)

    def process(i, buf, cb, rlow, end):
        valids = []
        for j in range(CROWS):
            gr = cb + j
            valids.append(jnp.logical_and(gr >= rlow, gr < end))
        for kk in range(8):
            cur = res_v[i, pl.ds(kk * 16, 16)]
            for j in range(CROWS):
                v = buf[j, pl.ds(kk * 16, 16)]
                cur = jnp.maximum(cur, jnp.where(valids[j], v, NEG))
            res_v[i, pl.ds(kk * 16, 16)] = cur

    def seg_body(i, mybuf, mysem, nxtbuf, nxtsem):
        ovec = off_v[pl.ds(i, 16)]
        start = ovec[0]
        end = ovec[1]
        nstart = off_v[pl.ds(i + 1, 16)][0]

        @pl.when(i + 1 < SEGW)
        def _():
            pltpu.async_copy(
                h_hbm.at[pl.ds(chunk0_addr(nstart), CROWS)], nxtbuf, nxtsem
            )

        for kk in range(8):
            res_v[i, pl.ds(kk * 16, 16)] = neg
        a0 = (start // 8) * 8
        cb0 = chunk0_addr(start)
        nchunk = (end - a0 + CROWS - 1) // CROWS
        pltpu.make_async_copy(
            h_hbm.at[pl.ds(0, CROWS)], mybuf, mysem
        ).wait()
        process(i, mybuf, cb0, start, end)

        def tail(t, _):
            pos = a0 + CROWS * t
            cb = pl.multiple_of(jnp.minimum(pos, N - CROWS), 8)
            pltpu.sync_copy(h_hbm.at[pl.ds(cb, CROWS)], buf_t)
            process(i, buf_t, cb, jnp.maximum(start, pos), end)
            return 0

        lax.fori_loop(1, nchunk, tail, 0)

    s0 = off_v[pl.ds(0, 16)][0]
    pltpu.async_copy(h_hbm.at[pl.ds(chunk0_addr(s0), CROWS)], buf_a, sem_a)

    def pair(p, _):
        seg_body(2 * p, buf_a, sem_a, buf_b, sem_b)
        seg_body(2 * p + 1, buf_b, sem_b, buf_a, sem_a)
        return 0

    lax.fori_loop(0, SEGW // 2, pair, 0)

    @pl.when(wid < NWORK - 1)
    def _():
        pltpu.sync_copy(res_v, sm_hbm.at[pl.ds(lo, SEGW)])

    @pl.when(wid == NWORK - 1)
    def _():
        pltpu.sync_copy(res_v.at[pl.ds(0, 80)], sm_hbm.at[pl.ds(lo, 80)])


def _sc_segmax(h, off_padded):
    fn = functools.partial(
        pl.kernel,
        out_type=jax.ShapeDtypeStruct((NSEG, D), jnp.float32),
        mesh=plsc.VectorSubcoreMesh(**_SC_MESH),
        scratch_types=[
            pltpu.VMEM((344,), jnp.int32),
            pltpu.VMEM((SEGW, D), jnp.float32),
            pltpu.VMEM((CROWS, D), jnp.float32),
            pltpu.VMEM((CROWS, D), jnp.float32),
            pltpu.VMEM((CROWS, D), jnp.float32),
            pltpu.SemaphoreType.DMA,
            pltpu.SemaphoreType.DMA,
        ],
    )(_segmax_kernel)
    return fn(h, off_padded)


def _make_gather(nrows, idx_off):
    rpw = nrows // NWORK
    n = rpw // GCH

    def gather_kernel(tab_hbm, ids_hbm, out_hbm, ids_v,
                      rows_a, rows_b, sem_a, sem_b, sem_wa, sem_wb):
        wid = _wid()
        base = wid * rpw
        pltpu.sync_copy(ids_hbm.at[pl.ds(base + idx_off, rpw)], ids_v)

        def issue(t, buf, sem):
            pltpu.async_copy(tab_hbm.at[ids_v.at[pl.ds(t * GCH, GCH)]],
                             buf, sem)

        def wait(buf, sem):
            pltpu.make_async_copy(
                tab_hbm.at[ids_v.at[pl.ds(0, GCH)]], buf, sem
            ).wait()

        def wait_w(buf, sem):
            pltpu.make_async_copy(
                buf, out_hbm.at[pl.ds(base, GCH)], sem
            ).wait()

        issue(0, rows_a, sem_a)

        def pair(p, _):
            t0 = 2 * p
            t1 = t0 + 1
            wait(rows_a, sem_a)

            @pl.when(t1 < n)
            def _():
                @pl.when(p > 0)
                def _():
                    wait_w(rows_b, sem_wb)

                issue(t1, rows_b, sem_b)

            pltpu.async_copy(rows_a, out_hbm.at[pl.ds(base + t0 * GCH, GCH)],
                             sem_wa)

            @pl.when(t1 < n)
            def _():
                @pl.when(t1 + 1 < n)
                def _():
                    wait_w(rows_a, sem_wa)
                    issue(t1 + 1, rows_a, sem_a)

                wait(rows_b, sem_b)
                pltpu.async_copy(rows_b,
                                 out_hbm.at[pl.ds(base + t1 * GCH, GCH)],
                                 sem_wb)

            return 0

        lax.fori_loop(0, (n + 1) // 2, pair, 0)
        wait_w(rows_a, sem_wa)
        wait_w(rows_b, sem_wb)

    def run(tab, ids):
        fn = functools.partial(
            pl.kernel,
            out_type=jax.ShapeDtypeStruct((nrows, D), jnp.float32),
            mesh=plsc.VectorSubcoreMesh(**_SC_MESH),
            scratch_types=[
                pltpu.VMEM((rpw,), jnp.int32),
                pltpu.VMEM((GCH, D), jnp.float32),
                pltpu.VMEM((GCH, D), jnp.float32),
                pltpu.SemaphoreType.DMA,
                pltpu.SemaphoreType.DMA,
                pltpu.SemaphoreType.DMA,
                pltpu.SemaphoreType.DMA,
            ],
        )(gather_kernel)
        return fn(tab, ids)

    return run


_gather_rows = _make_gather(N, 8)
_gather_uniq = _make_gather(UPAD, 0)


def kernel(x, polyline_ids, W0, b0, g0, be0, W1, b1, g1, be1, W2, b2, g2, be2):
    ids = polyline_ids.astype(jnp.int32)
    ids_pad = jnp.concatenate([jnp.full((8,), -1, jnp.int32), ids])

    sidx, spos, cnt = _sc_scan(ids_pad)
    off_padded, uniq_pad = _sc_merge(sidx, spos, cnt)

    h0 = _enc0(x, W0, b0, g0, be0)
    sm0 = _sc_segmax(h0, off_padded)
    c1 = _tiny(sm0, W1[D:], b1)
    a1 = _gather_rows(c1, ids_pad)
    h1 = _enc2(h0, a1, W1[:D], g1, be1)
    sm1 = _sc_segmax(h1, off_padded)
    c2 = _tiny(sm1, W2[D:], b2)
    a2 = _gather_rows(c2, ids_pad)
    h2 = _enc2(h1, a2, W2[:D], g2, be2)
    smf = _sc_segmax(h2, off_padded)

    feats_pad = _gather_uniq(smf, uniq_pad)
    return (feats_pad[:NSEG], uniq_pad[:NSEG])
